# Initial kernel scaffold; baseline (speedup 1.0000x reference)
import jax
import jax.numpy as jnp
from jax.experimental import pallas as pl


def kernel(x, coeffs, importance):
    raise NotImplementedError("write your pallas kernel here")


# speedup vs baseline: 6.7947x; 6.7947x over previous
import functools

import jax
import jax.numpy as jnp
from jax.experimental import pallas as pl
from jax.experimental.pallas import tpu as pltpu

_KEEP_RATIO = 0.5
_ROWS = 512


def _mask_kernel(keep, chunk, row_ref, col_ref, o_ref):
    D = row_ref.shape[1]
    row = row_ref[...]
    lane = jax.lax.broadcasted_iota(jnp.int32, (chunk, D), 1)
    for i in range(D // chunk):
        col = col_ref[pl.ds(i * chunk, chunk), :]
        qidx = jax.lax.broadcasted_iota(jnp.int32, (chunk, D), 0) + i * chunk
        gt = row > col
        tie = jnp.logical_and(row == col, lane < qidx)
        beat = jnp.where(jnp.logical_or(gt, tie), 1.0, 0.0)
        rank = jnp.sum(beat, axis=1, keepdims=True)
        o_ref[pl.ds(i * chunk, chunk), :] = jnp.where(rank < keep, 1.0, 0.0)


def _sumsq_kernel(nsteps, x_ref, mask_ref, o_ref, acc_ref):
    i = pl.program_id(0)
    xb = x_ref[...]
    part = jnp.sum(xb * xb, axis=0, keepdims=True)

    @pl.when(i == 0)
    def _init():
        acc_ref[...] = part

    @pl.when(i > 0)
    def _acc():
        acc_ref[...] = acc_ref[...] + part

    @pl.when(i == nsteps - 1)
    def _fin():
        o_ref[0, 0] = jnp.sum(acc_ref[...] * (1.0 - mask_ref[...]))


def _poly_kernel(x_ref, mask_ref, coef_ref, tot_ref, o_ref):
    x = x_ref[...]
    deg = coef_ref.shape[1]
    p = coef_ref[0, deg - 1]
    for k in range(deg - 2, -1, -1):
        p = p * x + coef_ref[0, k]
    loss = 1e-6 * jnp.sqrt(tot_ref[0, 0])
    mult = jnp.where(mask_ref[...] > 0.5, p, 1.0)
    o_ref[...] = x * mult + loss


def kernel(x, coeffs, importance):
    B, T, D = x.shape
    keep = max(1, int(D * _KEEP_RATIO))
    deg = coeffs.shape[0]
    xf = x.reshape(B * T, D)
    n = B * T
    nsteps = n // _ROWS

    mask_col = pl.pallas_call(
        functools.partial(_mask_kernel, keep, 256),
        out_shape=jax.ShapeDtypeStruct((D, 1), jnp.float32),
    )(importance.reshape(1, D), importance.reshape(D, 1))
    mask = mask_col.reshape(1, D)

    total = pl.pallas_call(
        functools.partial(_sumsq_kernel, nsteps),
        grid=(nsteps,),
        in_specs=[
            pl.BlockSpec((_ROWS, D), lambda i: (i, 0)),
            pl.BlockSpec((1, D), lambda i: (0, 0)),
        ],
        out_specs=pl.BlockSpec(memory_space=pltpu.SMEM),
        out_shape=jax.ShapeDtypeStruct((1, 1), jnp.float32),
        scratch_shapes=[pltpu.VMEM((1, D), jnp.float32)],
    )(xf, mask)

    y = pl.pallas_call(
        _poly_kernel,
        grid=(nsteps,),
        in_specs=[
            pl.BlockSpec((_ROWS, D), lambda i: (i, 0)),
            pl.BlockSpec((1, D), lambda i: (0, 0)),
            pl.BlockSpec(memory_space=pltpu.SMEM),
            pl.BlockSpec(memory_space=pltpu.SMEM),
        ],
        out_specs=pl.BlockSpec((_ROWS, D), lambda i: (i, 0)),
        out_shape=jax.ShapeDtypeStruct((n, D), jnp.float32),
        compiler_params=pltpu.CompilerParams(
            dimension_semantics=("parallel",),
        ),
    )(xf, mask, coeffs.reshape(1, deg), total)

    return y.reshape(B, T, D)


# speedup vs baseline: 7.7972x; 1.1475x over previous
import functools

import jax
import jax.numpy as jnp
from jax.experimental import pallas as pl
from jax.experimental.pallas import tpu as pltpu

_KEEP_RATIO = 0.5
_ROWS = 512
_CHUNK = 256


def _fused_kernel(keep, nsteps, row_ref, col_ref, x_ref, coef_ref,
                  o_ref, acc_ref, mask_ref, loss_ref):
    i = pl.program_id(0)
    D = row_ref.shape[1]

    @pl.when(i == 0)
    def _mask_and_init():
        row = row_ref[...]
        kidx0 = jax.lax.broadcasted_iota(jnp.int32, (_CHUNK, D), 0)
        qidx = jax.lax.broadcasted_iota(jnp.int32, (_CHUNK, D), 1)
        rank = jnp.zeros((1, D), jnp.float32)
        for c in range(D // _CHUNK):
            col = col_ref[pl.ds(c * _CHUNK, _CHUNK), :]
            kidx = kidx0 + c * _CHUNK
            beat = jnp.logical_or(
                col > row,
                jnp.logical_and(col == row, kidx < qidx))
            rank = rank + jnp.sum(jnp.where(beat, 1.0, 0.0),
                                  axis=0, keepdims=True)
        mask_ref[...] = jnp.where(rank < keep, 1.0, 0.0)
        acc_ref[...] = jnp.zeros((1, D), jnp.float32)

    @pl.when(i < nsteps)
    def _phase1():
        xb = x_ref[...]
        acc_ref[...] = acc_ref[...] + jnp.sum(xb * xb, axis=0, keepdims=True)

    @pl.when(i == nsteps - 1)
    def _loss():
        loss_ref[0, 0] = 1e-6 * jnp.sqrt(
            jnp.sum(acc_ref[...] * (1.0 - mask_ref[...])))

    @pl.when(i >= nsteps)
    def _phase2():
        x = x_ref[...]
        deg = coef_ref.shape[1]
        p = coef_ref[0, deg - 1]
        for k in range(deg - 2, -1, -1):
            p = p * x + coef_ref[0, k]
        mult = jnp.where(mask_ref[...] > 0.5, p, 1.0)
        o_ref[...] = x * mult + loss_ref[0, 0]


def kernel(x, coeffs, importance):
    B, T, D = x.shape
    keep = max(1, int(D * _KEEP_RATIO))
    deg = coeffs.shape[0]
    n = B * T
    nsteps = n // _ROWS
    xf = x.reshape(n, D)

    y = pl.pallas_call(
        functools.partial(_fused_kernel, keep, nsteps),
        grid=(2 * nsteps,),
        in_specs=[
            pl.BlockSpec((1, D), lambda i: (0, 0)),
            pl.BlockSpec((D, 1), lambda i: (0, 0)),
            pl.BlockSpec((_ROWS, D), lambda i: (i % nsteps, 0)),
            pl.BlockSpec(memory_space=pltpu.SMEM),
        ],
        out_specs=pl.BlockSpec((_ROWS, D), lambda i: (jnp.maximum(i - nsteps, 0), 0)),
        out_shape=jax.ShapeDtypeStruct((n, D), jnp.float32),
        scratch_shapes=[
            pltpu.VMEM((1, D), jnp.float32),
            pltpu.VMEM((1, D), jnp.float32),
            pltpu.SMEM((1, 1), jnp.float32),
        ],
    )(importance.reshape(1, D), importance.reshape(D, 1), xf,
      coeffs.reshape(1, deg))

    return y.reshape(B, T, D)


# speedup vs baseline: 8.1907x; 1.0505x over previous
import functools

import jax
import jax.numpy as jnp
from jax.experimental import pallas as pl
from jax.experimental.pallas import tpu as pltpu

_KEEP_RATIO = 0.5
_ROWS = 1024
_CHUNK = 256


def _fused_kernel(keep, nsteps, row_ref, col_ref, x_ref, coef_ref,
                  o_ref, acc_ref, mask_ref, loss_ref):
    i = pl.program_id(0)
    D = row_ref.shape[1]

    @pl.when(i == 0)
    def _mask_and_init():
        row = row_ref[...]
        kidx0 = jax.lax.broadcasted_iota(jnp.int32, (_CHUNK, D), 0)
        qidx = jax.lax.broadcasted_iota(jnp.int32, (_CHUNK, D), 1)
        rank = jnp.zeros((1, D), jnp.float32)
        for c in range(D // _CHUNK):
            col = col_ref[pl.ds(c * _CHUNK, _CHUNK), :]
            kidx = kidx0 + c * _CHUNK
            beat = jnp.logical_or(
                col > row,
                jnp.logical_and(col == row, kidx < qidx))
            rank = rank + jnp.sum(jnp.where(beat, 1.0, 0.0),
                                  axis=0, keepdims=True)
        mask_ref[...] = jnp.where(rank < keep, 1.0, 0.0)
        acc_ref[...] = jnp.zeros((1, D), jnp.float32)

    @pl.when(i < nsteps)
    def _phase1():
        xb = x_ref[...]
        acc_ref[...] = acc_ref[...] + jnp.sum(xb * xb, axis=0, keepdims=True)

    @pl.when(i == nsteps - 1)
    def _loss():
        loss_ref[0, 0] = 1e-6 * jnp.sqrt(
            jnp.sum(acc_ref[...] * (1.0 - mask_ref[...])))

    @pl.when(i >= nsteps)
    def _phase2():
        x = x_ref[...]
        deg = coef_ref.shape[1]
        p = coef_ref[0, deg - 1]
        for k in range(deg - 2, -1, -1):
            p = p * x + coef_ref[0, k]
        mult = jnp.where(mask_ref[...] > 0.5, p, 1.0)
        o_ref[...] = x * mult + loss_ref[0, 0]


def kernel(x, coeffs, importance):
    B, T, D = x.shape
    keep = max(1, int(D * _KEEP_RATIO))
    deg = coeffs.shape[0]
    n = B * T
    nsteps = n // _ROWS
    xf = x.reshape(n, D)

    y = pl.pallas_call(
        functools.partial(_fused_kernel, keep, nsteps),
        grid=(2 * nsteps,),
        in_specs=[
            pl.BlockSpec((1, D), lambda i: (0, 0)),
            pl.BlockSpec((D, 1), lambda i: (0, 0)),
            pl.BlockSpec((_ROWS, D), lambda i: (i % nsteps, 0)),
            pl.BlockSpec(memory_space=pltpu.SMEM),
        ],
        out_specs=pl.BlockSpec((_ROWS, D), lambda i: (jnp.maximum(i - nsteps, 0), 0)),
        out_shape=jax.ShapeDtypeStruct((n, D), jnp.float32),
        scratch_shapes=[
            pltpu.VMEM((1, D), jnp.float32),
            pltpu.VMEM((1, D), jnp.float32),
            pltpu.SMEM((1, 1), jnp.float32),
        ],
    )(importance.reshape(1, D), importance.reshape(D, 1), xf,
      coeffs.reshape(1, deg))

    return y.reshape(B, T, D)


# speedup vs baseline: 8.2407x; 1.0061x over previous
import functools

import jax
import jax.numpy as jnp
from jax.experimental import pallas as pl
from jax.experimental.pallas import tpu as pltpu

_KEEP_RATIO = 0.5
_ROWS = 1024
_CHUNK = 256


def _fused_kernel(keep, nsteps, row_ref, col_ref, x_ref, coef_ref,
                  o_ref, acc_ref, mask_ref, ab_ref, loss_ref):
    i = pl.program_id(0)
    D = row_ref.shape[1]
    deg = coef_ref.shape[1]

    @pl.when(i == 0)
    def _mask_and_init():
        row = row_ref[...]
        kidx0 = jax.lax.broadcasted_iota(jnp.int32, (_CHUNK, D), 0)
        qidx = jax.lax.broadcasted_iota(jnp.int32, (_CHUNK, D), 1)
        rank = jnp.zeros((1, D), jnp.float32)
        for c in range(D // _CHUNK):
            col = col_ref[pl.ds(c * _CHUNK, _CHUNK), :]
            kidx = kidx0 + c * _CHUNK
            beat = jnp.logical_or(
                col > row,
                jnp.logical_and(col == row, kidx < qidx))
            rank = rank + jnp.sum(jnp.where(beat, 1.0, 0.0),
                                  axis=0, keepdims=True)
        m = rank < keep
        mask_ref[...] = jnp.where(m, 1.0, 0.0)
        for k in range(deg):
            ab_ref[k:k + 1, :] = jnp.where(
                m, coef_ref[0, k], 1.0 if k == 0 else 0.0)
        acc_ref[...] = jnp.zeros((1, D), jnp.float32)

    @pl.when(i < nsteps)
    def _phase1():
        xb = x_ref[...]
        acc_ref[...] = acc_ref[...] + jnp.sum(xb * xb, axis=0, keepdims=True)

    @pl.when(i == nsteps - 1)
    def _loss():
        loss_ref[0, 0] = 1e-6 * jnp.sqrt(
            jnp.sum(acc_ref[...] * (1.0 - mask_ref[...])))

    @pl.when(i >= nsteps)
    def _phase2():
        x = x_ref[...]
        p = ab_ref[deg - 1:deg, :] * x
        for k in range(deg - 2, -1, -1):
            p = (p + ab_ref[k:k + 1, :]) * x
        o_ref[...] = p + loss_ref[0, 0]


def kernel(x, coeffs, importance):
    B, T, D = x.shape
    keep = max(1, int(D * _KEEP_RATIO))
    deg = coeffs.shape[0]
    n = B * T
    nsteps = n // _ROWS
    xf = x.reshape(n, D)

    y = pl.pallas_call(
        functools.partial(_fused_kernel, keep, nsteps),
        grid=(2 * nsteps,),
        in_specs=[
            pl.BlockSpec((1, D), lambda i: (0, 0)),
            pl.BlockSpec((D, 1), lambda i: (0, 0)),
            pl.BlockSpec((_ROWS, D), lambda i: (i % nsteps, 0)),
            pl.BlockSpec(memory_space=pltpu.SMEM),
        ],
        out_specs=pl.BlockSpec((_ROWS, D), lambda i: (jnp.maximum(i - nsteps, 0), 0)),
        out_shape=jax.ShapeDtypeStruct((n, D), jnp.float32),
        scratch_shapes=[
            pltpu.VMEM((1, D), jnp.float32),
            pltpu.VMEM((1, D), jnp.float32),
            pltpu.VMEM((deg, D), jnp.float32),
            pltpu.SMEM((1, 1), jnp.float32),
        ],
    )(importance.reshape(1, D), importance.reshape(D, 1), xf,
      coeffs.reshape(1, deg))

    return y.reshape(B, T, D)
